# baseline (speedup 1.0000x reference)
import functools

import jax
import jax.numpy as jnp
import numpy as np
from jax import lax
from jax.experimental import pallas as pl
from jax.experimental.pallas import tpu as pltpu
from jax.experimental.pallas import tpu_sc as plsc

F = 26
E = 16
A = 16
B = 4096
FIELD_DIM = 100000
TOTAL = F * FIELD_DIM
_OFFSETS = np.arange(F, dtype=np.int32) * FIELD_DIM

NC, NS = 2, 16
NW = NC * NS
BF = B * F
PER_W = BF // NW
LANES = 128
CHUNKS = PER_W // LANES
RPL = 128 // E
LIN_ROWS = (TOTAL + 127) // 128


def _gather_body(tblr, linr, idx8_hbm, rem8_hbm, idxl_hbm, reml_hbm,
                 emb_out, lin_out, idx8_v, rem8_v, idxl_v, reml_v,
                 stage0, stage1, stage2, stage3, tmp_v, lv_v,
                 sem0, sem1, sem2, sem3):
    wid = lax.axis_index("s") * NC + lax.axis_index("c")
    pltpu.sync_copy(idx8_hbm.at[wid], idx8_v)
    pltpu.sync_copy(rem8_hbm.at[wid], rem8_v)
    pltpu.sync_copy(idxl_hbm.at[wid], idxl_v)
    pltpu.sync_copy(reml_hbm.at[wid], reml_v)
    iota16 = lax.broadcasted_iota(jnp.int32, (16,), 0)

    def extract_emb(j, stage):
        def group(g, c2):
            rows16 = iota16 + g * 16
            rem16 = rem8_v[j, pl.ds(g * 16, 16)]
            lanebase = rem16 * 16
            for e in range(E):
                v = plsc.load_gather(stage, [rows16, lanebase + e])
                plsc.store_scatter(tmp_v, [rows16, jnp.full((16,), e, jnp.int32)], v)
            return c2

        lax.fori_loop(0, LANES // 16, group, 0, unroll=False)
        pltpu.sync_copy(tmp_v, emb_out.at[wid, j])

    def extract_lin(j, stage):
        def group(g, c2):
            rows16 = iota16 + g * 16
            rem16 = reml_v[j, pl.ds(g * 16, 16)]
            v = plsc.load_gather(stage, [rows16, rem16])
            lv_v[j, pl.ds(g * 16, 16)] = v
            return c2

        lax.fori_loop(0, LANES // 16, group, 0, unroll=False)

    pltpu.async_copy(tblr.at[idx8_v.at[0]], stage0, sem0)
    pltpu.async_copy(linr.at[idxl_v.at[0]], stage2, sem2)

    def pair(t, carry):
        j0 = 2 * t
        pltpu.async_copy(tblr.at[idx8_v.at[j0 + 1]], stage1, sem1)
        pltpu.async_copy(linr.at[idxl_v.at[j0 + 1]], stage3, sem3)
        pltpu.make_async_copy(tblr.at[idx8_v.at[j0]], stage0, sem0).wait()
        extract_emb(j0, stage0)

        @pl.when(j0 + 2 < CHUNKS)
        def _():
            pltpu.async_copy(tblr.at[idx8_v.at[j0 + 2]], stage0, sem0)

        pltpu.make_async_copy(linr.at[idxl_v.at[j0]], stage2, sem2).wait()
        extract_lin(j0, stage2)

        @pl.when(j0 + 2 < CHUNKS)
        def _():
            pltpu.async_copy(linr.at[idxl_v.at[j0 + 2]], stage2, sem2)

        pltpu.make_async_copy(tblr.at[idx8_v.at[j0 + 1]], stage1, sem1).wait()
        extract_emb(j0 + 1, stage1)
        pltpu.make_async_copy(linr.at[idxl_v.at[j0 + 1]], stage3, sem3).wait()
        extract_lin(j0 + 1, stage3)
        return carry

    lax.fori_loop(0, CHUNKS // 2, pair, 0, unroll=False)
    pltpu.sync_copy(lv_v, lin_out.at[wid])


def _sc_gather(tableR, linR, idx8, rem8, idxl, reml):
    mesh = plsc.VectorSubcoreMesh(core_axis_name="c", subcore_axis_name="s",
                                  num_cores=NC, num_subcores=NS)
    run = functools.partial(
        pl.kernel,
        out_type=[
            jax.ShapeDtypeStruct((NW, CHUNKS, LANES, E), jnp.float32),
            jax.ShapeDtypeStruct((NW, CHUNKS, LANES), jnp.float32),
        ],
        mesh=mesh,
        scratch_types=[
            pltpu.VMEM((CHUNKS, LANES), jnp.int32),
            pltpu.VMEM((CHUNKS, LANES), jnp.int32),
            pltpu.VMEM((CHUNKS, LANES), jnp.int32),
            pltpu.VMEM((CHUNKS, LANES), jnp.int32),
            pltpu.VMEM((LANES, 128), jnp.float32),
            pltpu.VMEM((LANES, 128), jnp.float32),
            pltpu.VMEM((LANES, 128), jnp.float32),
            pltpu.VMEM((LANES, 128), jnp.float32),
            pltpu.VMEM((LANES, E), jnp.float32),
            pltpu.VMEM((CHUNKS, LANES), jnp.float32),
            pltpu.SemaphoreType.DMA,
            pltpu.SemaphoreType.DMA,
            pltpu.SemaphoreType.DMA,
            pltpu.SemaphoreType.DMA,
        ],
        compiler_params=pltpu.CompilerParams(needs_layout_passes=False),
    )(_gather_body)
    return run(tableR, linR, idx8, rem8, idxl, reml)


WROWS = 1664
WLINES = WROWS // RPL
NFULL = 1562
SROWS = 768
SLINES = SROWS // RPL
NWIN = NFULL + 1
WPW = (NWIN + NW - 1) // NW
R_LINES = TOTAL // RPL


def _repack_sc_body(tt, tail, out_hbm, buf, lines_v, sem):
    wid = lax.axis_index("s") * NC + lax.axis_index("c")
    iota16 = lax.broadcasted_iota(jnp.int32, (16,), 0)

    @pl.when(wid == 0)
    def _():
        pltpu.sync_copy(tail, out_hbm.at[pl.ds(R_LINES - RPL, RPL)])

    def extract(nlines):
        def gblock(g0, c2):
            gs = g0 * 16 + iota16
            for e in range(E):
                for s in range(RPL):
                    v = plsc.load_gather(buf, [jnp.full((16,), e, jnp.int32),
                                               gs * RPL + s])
                    plsc.store_scatter(lines_v, [gs, jnp.full((16,), s * E + e, jnp.int32)], v)
            return c2

        lax.fori_loop(0, nlines // 16, gblock, 0, unroll=False)

    def window(k, carry):
        win = wid * WPW + k

        @pl.when(win < NFULL)
        def _():
            c0 = pltpu.async_copy(tt.at[pl.ds(0, 8), pl.ds(win * WROWS, WROWS)],
                                  buf.at[pl.ds(0, 8)], sem)
            c1 = pltpu.async_copy(tt.at[pl.ds(8, 8), pl.ds(win * WROWS, WROWS)],
                                  buf.at[pl.ds(8, 8)], sem)
            c0.wait()
            c1.wait()
            extract(WLINES)
            pltpu.sync_copy(lines_v, out_hbm.at[pl.ds(win * WLINES, WLINES)])

        @pl.when(win == NFULL)
        def _():
            c0 = pltpu.async_copy(tt.at[pl.ds(0, 8), pl.ds(NFULL * WROWS, SROWS)],
                                  buf.at[pl.ds(0, 8), pl.ds(0, SROWS)], sem)
            c1 = pltpu.async_copy(tt.at[pl.ds(8, 8), pl.ds(NFULL * WROWS, SROWS)],
                                  buf.at[pl.ds(8, 8), pl.ds(0, SROWS)], sem)
            c0.wait()
            c1.wait()
            extract(SLINES)
            pltpu.sync_copy(lines_v.at[pl.ds(0, SLINES)],
                            out_hbm.at[pl.ds(NFULL * WLINES, SLINES)])

        return carry

    lax.fori_loop(0, WPW, window, 0, unroll=False)


def _repack_tc(tableT, tail_lines):
    mesh = plsc.VectorSubcoreMesh(core_axis_name="c", subcore_axis_name="s",
                                  num_cores=NC, num_subcores=NS)
    run = functools.partial(
        pl.kernel,
        out_type=[jax.ShapeDtypeStruct((R_LINES, 128), jnp.float32)],
        mesh=mesh,
        scratch_types=[
            pltpu.VMEM((E, WROWS), jnp.float32),
            pltpu.VMEM((WLINES, 128), jnp.float32),
            pltpu.SemaphoreType.DMA,
        ],
        compiler_params=pltpu.CompilerParams(needs_layout_passes=False),
    )(_repack_sc_body)
    return run(tableT, tail_lines)[0]


BT = 128
FP = 32
W = FP * E


def _afm_body(embp_ref, embr_ref, linv_ref, t16_ref, w32_ref, ab_ref,
              pv_ref, t32_ref, fct_ref, consts_ref, out_ref):
    bf = jnp.bfloat16
    xp = embp_ref[...].astype(bf)
    r2 = jnp.dot(embr_ref[...].astype(bf), t16_ref[...],
                 preferred_element_type=jnp.float32).astype(bf)
    inner = r2.reshape(BT, F, W) * xp[:, None, :]
    inner2 = inner.reshape(BT * F, W)
    att = jnp.dot(inner2, w32_ref[...], preferred_element_type=jnp.float32)
    att = jnp.maximum(att + ab_ref[...], 0.0)
    logits = jnp.dot(att.astype(bf), pv_ref[...],
                     preferred_element_type=jnp.float32)
    logits = logits + consts_ref[0, 0]
    lg3 = logits.reshape(BT, F, FP)
    ii = lax.broadcasted_iota(jnp.int32, (F, FP), 0)
    jj = lax.broadcasted_iota(jnp.int32, (F, FP), 1)
    mask = ((jj > ii) & (jj < F))[None]
    lg3 = jnp.where(mask, lg3, -1e30)
    m = jnp.max(jnp.max(lg3, axis=2), axis=1)
    ex = jnp.where(mask, jnp.exp(lg3 - m[:, None, None]), 0.0)
    s = jnp.sum(jnp.sum(ex, axis=2), axis=1)
    scores = (ex / s[:, None, None]).reshape(BT * F, FP)
    srep = jnp.dot(scores.astype(bf), t32_ref[...],
                   preferred_element_type=jnp.float32).astype(bf)
    ws = (srep * inner2).astype(jnp.float32).reshape(BT, F, W)
    sums = jnp.sum(ws, axis=1)
    afm = jnp.dot(sums.astype(bf), fct_ref[...].astype(bf),
                  preferred_element_type=jnp.float32)
    lin = jnp.sum(linv_ref[...], axis=1, keepdims=True)
    out_ref[...] = lin + afm + consts_ref[0, 1]


def _afm_tc(embp, embr, linv, t16, w32, abt, pv, t32, fct, consts):
    rep = lambda i: (0, 0)
    return pl.pallas_call(
        _afm_body,
        grid=(B // BT,),
        in_specs=[
            pl.BlockSpec((BT, W), lambda i: (i, 0)),
            pl.BlockSpec((BT * F, E), lambda i: (i, 0)),
            pl.BlockSpec((BT, F), lambda i: (i, 0)),
            pl.BlockSpec((E, W), rep),
            pl.BlockSpec((W, W), rep),
            pl.BlockSpec((1, W), rep),
            pl.BlockSpec((W, FP), rep),
            pl.BlockSpec((FP, W), rep),
            pl.BlockSpec((W, 1), rep),
            pl.BlockSpec((1, 2), rep),
        ],
        out_specs=pl.BlockSpec((BT, 1), lambda i: (i, 0)),
        out_shape=jax.ShapeDtypeStruct((B, 1), jnp.float32),
    )(embp, embr, linv, t16, w32, abt, pv, t32, fct, consts)


def kernel(x, table, linear_w, bias, attn_W, attn_b, proj_W, proj_b, fc_W, fc_b):
    tail_lines = lax.slice(table, (TOTAL - 64, 0), (TOTAL, E)).reshape(RPL, 128)
    tableR = _repack_tc(table.T, tail_lines)
    lin_flat = jnp.concatenate(
        [linear_w[:, 0], jnp.zeros((LIN_ROWS * 128 - TOTAL,), jnp.float32)])
    linR = lin_flat.reshape(LIN_ROWS, 128)
    idx = x + jnp.asarray(_OFFSETS)[None, :]
    idx8 = (idx >> 3).reshape(NW, CHUNKS, LANES)
    rem8 = (idx & 7).reshape(NW, CHUNKS, LANES)
    idxl = (idx >> 7).reshape(NW, CHUNKS, LANES)
    reml = (idx & 127).reshape(NW, CHUNKS, LANES)
    emb4, lin3 = _sc_gather(tableR, linR, idx8, rem8, idxl, reml)

    embr = emb4.reshape(B * F, E)
    embp = jnp.pad(emb4.reshape(B, F * E), ((0, 0), (0, (FP - F) * E)))
    linv = lin3.reshape(B, F)
    bf = jnp.bfloat16
    eye32 = jnp.eye(FP, dtype=jnp.float32)
    t16 = jnp.tile(jnp.eye(E, dtype=jnp.float32), (1, FP)).astype(bf)
    w32 = jnp.kron(eye32, attn_W).astype(bf)
    abt = jnp.tile(attn_b, FP).reshape(1, W)
    pv = jnp.kron(eye32, proj_W).astype(bf)
    t32 = jnp.kron(eye32, jnp.ones((1, E), jnp.float32)).astype(bf)
    fmask = (jnp.arange(FP) < F).astype(jnp.float32).reshape(FP, 1)
    fct = jnp.kron(fmask, fc_W)
    consts = jnp.stack([proj_b[0], bias[0] + fc_b[0]]).reshape(1, 2)
    return _afm_tc(embp, embr, linv, t16, w32, abt, pv, t32, fct, consts)


# speedup vs baseline: 1.2191x; 1.2191x over previous
import functools

import jax
import jax.numpy as jnp
import numpy as np
from jax import lax
from jax.experimental import pallas as pl
from jax.experimental.pallas import tpu as pltpu
from jax.experimental.pallas import tpu_sc as plsc

F = 26
E = 16
A = 16
B = 4096
FIELD_DIM = 100000
TOTAL = F * FIELD_DIM
_OFFSETS = np.arange(F, dtype=np.int32) * FIELD_DIM

NC, NS = 2, 16
NW = NC * NS
BF = B * F
PER_W = BF // NW
LANES = 128
CHUNKS = PER_W // LANES
RPL = 128 // E
LIN_ROWS = (TOTAL + 127) // 128


def _gather_body(tblr, linr, idx8_hbm, rem8_hbm, idxl_hbm, reml_hbm,
                 emb_out, lin_out, idx8_v, rem8_v, idxl_v, reml_v,
                 stage0, stage1, stage2, stage3, tmp_v, lv_v,
                 sem0, sem1, sem2, sem3):
    wid = lax.axis_index("s") * NC + lax.axis_index("c")
    pltpu.sync_copy(idx8_hbm.at[wid], idx8_v)
    pltpu.sync_copy(rem8_hbm.at[wid], rem8_v)
    pltpu.sync_copy(idxl_hbm.at[wid], idxl_v)
    pltpu.sync_copy(reml_hbm.at[wid], reml_v)
    iota16 = lax.broadcasted_iota(jnp.int32, (16,), 0)

    def extract_emb(j, stage):
        def group(g, c2):
            rows16 = iota16 + g * 16
            rem16 = rem8_v[j, pl.ds(g * 16, 16)]
            lanebase = rem16 * 16
            for e in range(E):
                v = plsc.load_gather(stage, [rows16, lanebase + e])
                plsc.store_scatter(tmp_v, [rows16, jnp.full((16,), e, jnp.int32)], v)
            return c2

        lax.fori_loop(0, LANES // 16, group, 0, unroll=False)
        pltpu.sync_copy(tmp_v, emb_out.at[wid, j])

    def extract_lin(j, stage):
        def group(g, c2):
            rows16 = iota16 + g * 16
            rem16 = reml_v[j, pl.ds(g * 16, 16)]
            v = plsc.load_gather(stage, [rows16, rem16])
            lv_v[j, pl.ds(g * 16, 16)] = v
            return c2

        lax.fori_loop(0, LANES // 16, group, 0, unroll=False)

    pltpu.async_copy(tblr.at[idx8_v.at[0]], stage0, sem0)
    pltpu.async_copy(linr.at[idxl_v.at[0]], stage2, sem2)

    def pair(t, carry):
        j0 = 2 * t
        pltpu.async_copy(tblr.at[idx8_v.at[j0 + 1]], stage1, sem1)
        pltpu.async_copy(linr.at[idxl_v.at[j0 + 1]], stage3, sem3)
        pltpu.make_async_copy(tblr.at[idx8_v.at[j0]], stage0, sem0).wait()
        extract_emb(j0, stage0)

        @pl.when(j0 + 2 < CHUNKS)
        def _():
            pltpu.async_copy(tblr.at[idx8_v.at[j0 + 2]], stage0, sem0)

        pltpu.make_async_copy(linr.at[idxl_v.at[j0]], stage2, sem2).wait()
        extract_lin(j0, stage2)

        @pl.when(j0 + 2 < CHUNKS)
        def _():
            pltpu.async_copy(linr.at[idxl_v.at[j0 + 2]], stage2, sem2)

        pltpu.make_async_copy(tblr.at[idx8_v.at[j0 + 1]], stage1, sem1).wait()
        extract_emb(j0 + 1, stage1)
        pltpu.make_async_copy(linr.at[idxl_v.at[j0 + 1]], stage3, sem3).wait()
        extract_lin(j0 + 1, stage3)
        return carry

    lax.fori_loop(0, CHUNKS // 2, pair, 0, unroll=False)
    pltpu.sync_copy(lv_v, lin_out.at[wid])


def _sc_gather(tableR, linR, idx8, rem8, idxl, reml):
    mesh = plsc.VectorSubcoreMesh(core_axis_name="c", subcore_axis_name="s",
                                  num_cores=NC, num_subcores=NS)
    run = functools.partial(
        pl.kernel,
        out_type=[
            jax.ShapeDtypeStruct((NW, CHUNKS, LANES, E), jnp.float32),
            jax.ShapeDtypeStruct((NW, CHUNKS, LANES), jnp.float32),
        ],
        mesh=mesh,
        scratch_types=[
            pltpu.VMEM((CHUNKS, LANES), jnp.int32),
            pltpu.VMEM((CHUNKS, LANES), jnp.int32),
            pltpu.VMEM((CHUNKS, LANES), jnp.int32),
            pltpu.VMEM((CHUNKS, LANES), jnp.int32),
            pltpu.VMEM((LANES, 128), jnp.float32),
            pltpu.VMEM((LANES, 128), jnp.float32),
            pltpu.VMEM((LANES, 128), jnp.float32),
            pltpu.VMEM((LANES, 128), jnp.float32),
            pltpu.VMEM((LANES, E), jnp.float32),
            pltpu.VMEM((CHUNKS, LANES), jnp.float32),
            pltpu.SemaphoreType.DMA,
            pltpu.SemaphoreType.DMA,
            pltpu.SemaphoreType.DMA,
            pltpu.SemaphoreType.DMA,
        ],
        compiler_params=pltpu.CompilerParams(needs_layout_passes=False),
    )(_gather_body)
    return run(tableR, linR, idx8, rem8, idxl, reml)


RCH = 16384
RGRP = RCH // RPL
RSTEPS = (TOTAL + RCH - 1) // RCH
R_LINES = RSTEPS * RGRP


def _repack_body(xt_ref, out_ref):
    xt = jnp.transpose(xt_ref[...])
    out_ref[...] = jnp.concatenate(
        [xt[RGRP * s:RGRP * (s + 1), :] for s in range(RPL)], axis=1)


def _repack_tc(tableT):
    return pl.pallas_call(
        _repack_body,
        grid=(RSTEPS,),
        in_specs=[pl.BlockSpec((E, RCH), lambda i: (0, i))],
        out_specs=pl.BlockSpec((RGRP, 128), lambda i: (i, 0)),
        out_shape=jax.ShapeDtypeStruct((R_LINES, 128), jnp.float32),
    )(tableT)


BT = 128
FP = 32
W = FP * E


def _afm_body(embp_ref, embr_ref, linv_ref, t16_ref, w32_ref, ab_ref,
              pv_ref, t32_ref, fct_ref, consts_ref, out_ref):
    bf = jnp.bfloat16
    xp = embp_ref[...].astype(bf)
    r2 = jnp.dot(embr_ref[...].astype(bf), t16_ref[...],
                 preferred_element_type=jnp.float32).astype(bf)
    inner = r2.reshape(BT, F, W) * xp[:, None, :]
    inner2 = inner.reshape(BT * F, W)
    att = jnp.dot(inner2, w32_ref[...], preferred_element_type=jnp.float32)
    att = jnp.maximum(att + ab_ref[...], 0.0)
    logits = jnp.dot(att.astype(bf), pv_ref[...],
                     preferred_element_type=jnp.float32)
    logits = logits + consts_ref[0, 0]
    lg3 = logits.reshape(BT, F, FP)
    ii = lax.broadcasted_iota(jnp.int32, (F, FP), 0)
    jj = lax.broadcasted_iota(jnp.int32, (F, FP), 1)
    mask = ((jj > ii) & (jj < F))[None]
    lg3 = jnp.where(mask, lg3, -1e30)
    m = jnp.max(jnp.max(lg3, axis=2), axis=1)
    ex = jnp.where(mask, jnp.exp(lg3 - m[:, None, None]), 0.0)
    s = jnp.sum(jnp.sum(ex, axis=2), axis=1)
    scores = (ex / s[:, None, None]).reshape(BT * F, FP)
    srep = jnp.dot(scores.astype(bf), t32_ref[...],
                   preferred_element_type=jnp.float32).astype(bf)
    ws = (srep * inner2).astype(jnp.float32).reshape(BT, F, W)
    sums = jnp.sum(ws, axis=1)
    afm = jnp.dot(sums.astype(bf), fct_ref[...].astype(bf),
                  preferred_element_type=jnp.float32)
    lin = jnp.sum(linv_ref[...], axis=1, keepdims=True)
    out_ref[...] = lin + afm + consts_ref[0, 1]


def _afm_tc(embp, embr, linv, t16, w32, abt, pv, t32, fct, consts):
    rep = lambda i: (0, 0)
    return pl.pallas_call(
        _afm_body,
        grid=(B // BT,),
        in_specs=[
            pl.BlockSpec((BT, W), lambda i: (i, 0)),
            pl.BlockSpec((BT * F, E), lambda i: (i, 0)),
            pl.BlockSpec((BT, F), lambda i: (i, 0)),
            pl.BlockSpec((E, W), rep),
            pl.BlockSpec((W, W), rep),
            pl.BlockSpec((1, W), rep),
            pl.BlockSpec((W, FP), rep),
            pl.BlockSpec((FP, W), rep),
            pl.BlockSpec((W, 1), rep),
            pl.BlockSpec((1, 2), rep),
        ],
        out_specs=pl.BlockSpec((BT, 1), lambda i: (i, 0)),
        out_shape=jax.ShapeDtypeStruct((B, 1), jnp.float32),
    )(embp, embr, linv, t16, w32, abt, pv, t32, fct, consts)


def kernel(x, table, linear_w, bias, attn_W, attn_b, proj_W, proj_b, fc_W, fc_b):
    tableR = _repack_tc(table.T)
    lin_flat = jnp.concatenate(
        [linear_w[:, 0], jnp.zeros((LIN_ROWS * 128 - TOTAL,), jnp.float32)])
    linR = lin_flat.reshape(LIN_ROWS, 128)
    idx = x + jnp.asarray(_OFFSETS)[None, :]
    idx8 = ((idx // RCH) * RGRP + (idx % RGRP)).reshape(NW, CHUNKS, LANES)
    rem8 = ((idx % RCH) // RGRP).reshape(NW, CHUNKS, LANES)
    idxl = (idx >> 7).reshape(NW, CHUNKS, LANES)
    reml = (idx & 127).reshape(NW, CHUNKS, LANES)
    emb4, lin3 = _sc_gather(tableR, linR, idx8, rem8, idxl, reml)

    embr = emb4.reshape(B * F, E)
    embp = jnp.pad(emb4.reshape(B, F * E), ((0, 0), (0, (FP - F) * E)))
    linv = lin3.reshape(B, F)
    bf = jnp.bfloat16
    eye32 = jnp.eye(FP, dtype=jnp.float32)
    t16 = jnp.tile(jnp.eye(E, dtype=jnp.float32), (1, FP)).astype(bf)
    w32 = jnp.kron(eye32, attn_W).astype(bf)
    abt = jnp.tile(attn_b, FP).reshape(1, W)
    pv = jnp.kron(eye32, proj_W).astype(bf)
    t32 = jnp.kron(eye32, jnp.ones((1, E), jnp.float32)).astype(bf)
    fmask = (jnp.arange(FP) < F).astype(jnp.float32).reshape(FP, 1)
    fct = jnp.kron(fmask, fc_W)
    consts = jnp.stack([proj_b[0], bias[0] + fc_b[0]]).reshape(1, 2)
    return _afm_tc(embp, embr, linv, t16, w32, abt, pv, t32, fct, consts)
